# baseline (speedup 1.0000x reference)
import functools

import jax
import jax.numpy as jnp
from jax import lax
from jax.experimental import pallas as pl
from jax.experimental.pallas import tpu as pltpu
from jax.experimental.pallas import tpu_sc as plsc

_NUM_CLASSES = 1000
_HIDDEN = 1024
_BATCH = 16384
_K = 8

_info = plsc.get_sparse_core_info()
_NC, _NS, _L = _info.num_cores, _info.num_subcores, _info.num_lanes
_NW = _NC * _NS
_BPW = _BATCH // _NW
_CH = 32
_NBUF = 3
_NCHUNK = _BPW // _CH

_mesh = plsc.VectorSubcoreMesh(core_axis_name="c", subcore_axis_name="s")


@functools.partial(
    pl.kernel,
    mesh=_mesh,
    out_type=jax.ShapeDtypeStruct((_BATCH, _HIDDEN), jnp.float32),
    scratch_types=[
        pltpu.VMEM((_BPW,), jnp.int32),
        pltpu.VMEM((_BPW,), jnp.int32),
        pltpu.VMEM((_BPW,), jnp.int32),
        pltpu.VMEM((_NBUF, _CH, _HIDDEN), jnp.float32),
        pltpu.SemaphoreType.DMA,
        pltpu.SemaphoreType.DMA,
    ],
)
def _embed(lab_hbm, fd_hbm, table_hbm, out_hbm, lab_v, fd_v, idx_v, buf, gsem, osem):
    wid = lax.axis_index("s") * _NC + lax.axis_index("c")
    base = wid * _BPW
    rep_off = (wid % _K) * (_NUM_CLASSES + 1)
    pltpu.async_copy(lab_hbm.at[pl.ds(base, _BPW)], lab_v, gsem).wait()
    pltpu.async_copy(fd_hbm.at[pl.ds(base, _BPW)], fd_v, osem).wait()

    for i in range(_BPW // _L):
        sl = pl.ds(i * _L, _L)
        idx_v[sl] = jnp.where(fd_v[sl] == 1, _NUM_CLASSES, lab_v[sl]) + rep_off

    def gather(c):
        return pltpu.async_copy(
            table_hbm.at[idx_v.at[pl.ds(c * _CH, _CH)]], buf.at[c % _NBUF], gsem
        )

    gh = [None] * _NCHUNK
    wh = [None] * _NCHUNK
    for c in range(_NBUF - 1):
        gh[c] = gather(c)
    for c in range(_NCHUNK):
        nxt = c + _NBUF - 1
        if nxt < _NCHUNK:
            if nxt >= _NBUF:
                wh[nxt - _NBUF].wait()
            gh[nxt] = gather(nxt)
        gh[c].wait()
        wh[c] = pltpu.async_copy(
            buf.at[c % _NBUF], out_hbm.at[pl.ds(base + c * _CH, _CH)], osem
        )
    for c in range(_NCHUNK - _NBUF, _NCHUNK):
        wh[c].wait()


def kernel(labels, force_drop_ids, embedding_table):
    replicated = jnp.tile(embedding_table, (_K, 1))
    return _embed(
        labels.astype(jnp.int32),
        force_drop_ids.astype(jnp.int32),
        replicated,
    )


# speedup vs baseline: 1.7281x; 1.7281x over previous
import functools

import jax
import jax.numpy as jnp
from jax import lax
from jax.experimental import pallas as pl
from jax.experimental.pallas import tpu as pltpu
from jax.experimental.pallas import tpu_sc as plsc

_NUM_CLASSES = 1000
_HIDDEN = 1024
_BATCH = 16384

_info = plsc.get_sparse_core_info()
_NC, _NS, _L = _info.num_cores, _info.num_subcores, _info.num_lanes
_NW = _NC * _NS
_BPW = _BATCH // _NW
_CH = 32
_NBUF = 2
_NCHUNK = _BPW // _CH


_mesh = plsc.VectorSubcoreMesh(core_axis_name="c", subcore_axis_name="s")


@functools.partial(
    pl.kernel,
    mesh=_mesh,
    compiler_params=pltpu.CompilerParams(use_tc_tiling_on_sc=False),
    out_type=jax.ShapeDtypeStruct((_BATCH, _HIDDEN), jnp.float32),
    scratch_types=[
        pltpu.VMEM((_BPW,), jnp.int32),
        pltpu.VMEM((_BPW,), jnp.int32),
        pltpu.VMEM((_NBUF, _CH, _HIDDEN), jnp.float32),
        pltpu.VMEM((_CH,), jnp.int32),
        pltpu.VMEM((_CH,), jnp.int32),
        pltpu.VMEM_SHARED((_NUM_CLASSES + 1, _HIDDEN), jnp.float32),
        pltpu.SemaphoreType.DMA,
        pltpu.SemaphoreType.DMA,
    ],
)
def _embed(lab_hbm, fd_hbm, table_hbm, out_hbm, lab_v, fd_v, buf, ix0, ix1, table_sp, sem, osem):
    sid = lax.axis_index("s")
    wid = sid * _NC + lax.axis_index("c")
    base = wid * _BPW
    _ROWS_PER_TILE = (_NUM_CLASSES + 1 + _NS - 1) // _NS
    for t in range(_NS):
        start = t * _ROWS_PER_TILE
        cnt = min(_ROWS_PER_TILE, _NUM_CLASSES + 1 - start)
        @pl.when(sid == t)
        def _(start=start, cnt=cnt):
            pltpu.sync_copy(
                table_hbm.at[pl.ds(start, cnt)], table_sp.at[pl.ds(start, cnt)]
            )

    pltpu.async_copy(lab_hbm.at[pl.ds(base, _BPW)], lab_v, sem).wait()
    pltpu.async_copy(fd_hbm.at[pl.ds(base, _BPW)], fd_v, sem).wait()

    for i in range(_BPW // _L):
        sl = pl.ds(i * _L, _L)
        lab_v[sl] = jnp.where(fd_v[sl] == 1, _NUM_CLASSES, lab_v[sl])

    plsc.subcore_barrier()

    wh = [None] * _NCHUNK
    rh = [[None] * _CH for _ in range(_NCHUNK)]

    ix = [ix0, ix1]

    def pull_chunk(c):
        s = c % _NBUF
        for q in range(_CH // _L):
            ix[s][pl.ds(q * _L, _L)] = lab_v[pl.ds(c * _CH + q * _L, _L)]
        rh[c][0] = pltpu.async_copy(table_sp.at[ix[s]], buf.at[s], sem)

    for c in range(_NCHUNK):
        if c < _NBUF - 1:
            pull_chunk(c)
    for c in range(_NCHUNK):
        nxt = c + _NBUF - 1
        if nxt < _NCHUNK:
            if nxt >= _NBUF:
                wh[nxt - _NBUF].wait()
            pull_chunk(nxt)
        rh[c][0].wait()
        wh[c] = pltpu.async_copy(
            buf.at[c % _NBUF], out_hbm.at[pl.ds(base + c * _CH, _CH)], osem
        )
    for c in range(_NCHUNK - _NBUF, _NCHUNK):
        wh[c].wait()


def kernel(labels, force_drop_ids, embedding_table):
    return _embed(
        labels.astype(jnp.int32),
        force_drop_ids.astype(jnp.int32),
        embedding_table,
    )
